# Initial kernel scaffold; baseline (speedup 1.0000x reference)
import jax
import jax.numpy as jnp
from jax.experimental import pallas as pl


def kernel(constraint_features, variable_features, edge_index, edge_attr, params):
    raise NotImplementedError("write your pallas kernel here")


# speedup vs baseline: 3.1450x; 3.1450x over previous
import functools

import jax
import jax.numpy as jnp
from jax import lax
from jax.experimental import pallas as pl
from jax.experimental.pallas import tpu as pltpu
from jax.experimental.pallas import tpu_sc as plsc

NN = 50000
EE = 800000
H = 64
HH = 32
NSUB = 16
EPT = EE // NSUB
K = 80
CPS = 25
SPT = EPT // (CPS * K)
UR = 200
NUNITS = NN // UR
NROW = EE // K

@functools.lru_cache(maxsize=None)
def _make_edge_seg_kernel():
    mesh = plsc.VectorSubcoreMesh(core_axis_name="c", subcore_axis_name="s")
    return functools.partial(
        pl.kernel,
        out_type=jax.ShapeDtypeStruct((2, NN, HH), jnp.float32),
        mesh=mesh,
        scratch_types=[
            pltpu.VMEM((CPS * K,), jnp.int32),
            pltpu.VMEM((CPS * K,), jnp.int32),
            pltpu.VMEM((CPS, K), jnp.int32),
            pltpu.VMEM((CPS, K), jnp.float32),
            pltpu.VMEM((HH,), jnp.float32),
            pltpu.VMEM((K, HH), jnp.float32),
            pltpu.VMEM((K, HH), jnp.float32),
            pltpu.VMEM((K, HH), jnp.float32),
            pltpu.VMEM((UR, HH), jnp.float32),
            pltpu.VMEM_SHARED((NN, HH), jnp.float32),
            pltpu.SemaphoreType.DMA,
            pltpu.SemaphoreType.DMA,
        ],
        compiler_params=pltpu.CompilerParams(needs_layout_passes=False, use_tc_tiling_on_sc=False),
    )(_edge_seg_body)


def _edge_seg_body(tD, tS, dstg, srcg, agg3, ea3, wa, out,
                   dbuf, sbuf, aggbuf, eabuf, wabuf, Ab, Bb, Mb, Zb, acc,
                   sem_a, sem_b):
    c = lax.axis_index("c")
    s = lax.axis_index("s")
    zero16 = jnp.zeros((16,), jnp.float32)

    def zrow(i, carry):
        Zb[i, pl.ds(0, 16)] = zero16
        Zb[i, pl.ds(16, 16)] = zero16
        return carry
    lax.fori_loop(0, UR, zrow, 0)

    nunits = (NUNITS - s + NSUB - 1) // NSUB

    def zcp(i, carry):
        pltpu.sync_copy(Zb, acc.at[pl.ds((s + NSUB * i) * UR, UR)])
        return carry
    lax.fori_loop(0, nunits, zcp, 0)

    pltpu.sync_copy(wa.at[pl.ds(c * HH, HH)], wabuf)
    wa0 = wabuf[pl.ds(0, 16)]
    wa1 = wabuf[pl.ds(16, 16)]
    plsc.subcore_barrier()

    def superbody(sc, carry):
        off = c * EE + s * EPT + sc * (CPS * K)
        r = s * SPT + sc
        pltpu.sync_copy(dstg.at[pl.ds(off, CPS * K)], dbuf)
        pltpu.sync_copy(srcg.at[pl.ds(off, CPS * K)], sbuf)
        pltpu.sync_copy(agg3.at[r], aggbuf)
        pltpu.sync_copy(ea3.at[r], eabuf)

        def chunkbody(k, carry2):
            cpa = pltpu.async_copy(tD.at[dbuf.at[pl.ds(k * K, K)]], Ab, sem_a)
            cpb = pltpu.async_copy(tS.at[sbuf.at[pl.ds(k * K, K)]], Bb, sem_b)
            cpa.wait()
            cpb.wait()
            kvec = jnp.full((16,), k, jnp.int32)
            for g in range(K // 16):
                for e in range(16):
                    ei_ = g * 16 + e
                    eb = plsc.load_gather(
                        eabuf, [kvec, jnp.full((16,), ei_, jnp.int32)])
                    a0 = Ab[ei_, pl.ds(0, 16)]
                    a1 = Ab[ei_, pl.ds(16, 16)]
                    b0 = Bb[ei_, pl.ds(0, 16)]
                    b1 = Bb[ei_, pl.ds(16, 16)]
                    Mb[ei_, pl.ds(0, 16)] = jnp.maximum(a0 + b0 + eb * wa0, 0.0)
                    Mb[ei_, pl.ds(16, 16)] = jnp.maximum(a1 + b1 + eb * wa1, 0.0)
            pltpu.sync_copy(Mb, acc.at[aggbuf.at[k]], add=True)
            return carry2
        lax.fori_loop(0, CPS, chunkbody, 0)
        return carry
    lax.fori_loop(0, SPT, superbody, 0)

    plsc.subcore_barrier()

    def ocp(i, carry):
        rs = (s + NSUB * i) * UR
        pltpu.sync_copy(acc.at[pl.ds(rs, UR)], out.at[c, pl.ds(rs, UR)])
        return carry
    lax.fori_loop(0, nunits, ocp, 0)


_CW = 16


@functools.lru_cache(maxsize=None)
def _make_count_kernel():
    mesh = plsc.VectorSubcoreMesh(core_axis_name="c", subcore_axis_name="s")
    return functools.partial(
        pl.kernel,
        out_type=jax.ShapeDtypeStruct((2, NN, _CW), jnp.float32),
        mesh=mesh,
        scratch_types=[
            pltpu.VMEM((CPS, K), jnp.int32),
            pltpu.VMEM((K, _CW), jnp.float32),
            pltpu.VMEM((UR, _CW), jnp.float32),
            pltpu.VMEM_SHARED((NN, _CW), jnp.float32),
        ],
        compiler_params=pltpu.CompilerParams(needs_layout_passes=False, use_tc_tiling_on_sc=False),
    )(_count_body)


def _count_body(agg4, out, aggbuf, Ob, Zb, acc):
    c = lax.axis_index("c")
    s = lax.axis_index("s")
    zero16 = jnp.zeros((16,), jnp.float32)
    one16 = jnp.ones((16,), jnp.float32)

    def fill(i, carry):
        Zb[i, pl.ds(0, 16)] = zero16
        return carry
    lax.fori_loop(0, UR, fill, 0)

    def fillo(i, carry):
        Ob[i, pl.ds(0, 16)] = one16
        return carry
    lax.fori_loop(0, K, fillo, 0)

    nunits = (NUNITS - s + NSUB - 1) // NSUB

    def zcp(i, carry):
        pltpu.sync_copy(Zb, acc.at[pl.ds((s + NSUB * i) * UR, UR)])
        return carry
    lax.fori_loop(0, nunits, zcp, 0)
    plsc.subcore_barrier()

    def superbody(sc, carry):
        r = c * (NROW // CPS) + s * SPT + sc
        pltpu.sync_copy(agg4.at[r], aggbuf)

        def chunkbody(k, carry2):
            pltpu.sync_copy(Ob, acc.at[aggbuf.at[k]], add=True)
            return carry2
        lax.fori_loop(0, CPS, chunkbody, 0)
        return carry
    lax.fori_loop(0, SPT, superbody, 0)

    plsc.subcore_barrier()

    def ocp(i, carry):
        rs = (s + NSUB * i) * UR
        pltpu.sync_copy(acc.at[pl.ds(rs, UR)], out.at[c, pl.ds(rs, UR)])
        return carry
    lax.fori_loop(0, nunits, ocp, 0)


_RB = 1000
_NG = NN // _RB


def _full(shape):
    return pl.BlockSpec(shape, lambda i: (0,) * len(shape))


def _rows(width):
    return pl.BlockSpec((_RB, width), lambda i: (i, 0))


def _project(x, W, b, extras):
    din = x.shape[1]
    ne = len(extras)

    def body(*refs):
        x_ref, W_ref, b_ref = refs[:3]
        prefs = refs[3:3 + 2 * ne]
        outs = refs[3 + 2 * ne:]
        y = jnp.dot(x_ref[...], W_ref[...],
                    preferred_element_type=jnp.float32) + b_ref[...]
        outs[0][...] = y
        for t in range(ne):
            outs[1 + t][...] = jnp.dot(
                y, prefs[2 * t][...],
                preferred_element_type=jnp.float32) + prefs[2 * t + 1][...]

    in_specs = [_rows(din), _full((din, H)), _full((1, H))]
    args = [x, W, b.reshape(1, H)]
    for (P, p) in extras:
        in_specs += [_full((H, H)), _full((1, H))]
        args += [P, p.reshape(1, H)]
    out_shapes = tuple(jax.ShapeDtypeStruct((NN, H), jnp.float32)
                       for _ in range(1 + ne))
    out_specs = tuple(_rows(H) for _ in range(1 + ne))
    return pl.pallas_call(
        body, grid=(_NG,), in_specs=in_specs, out_specs=out_specs,
        out_shape=out_shapes)(*args)


def _node_update(x, seg, cnt, W2, b2, M1, bm1, g, bln, M2, bm2, extras):
    ne = len(extras)

    def body(*refs):
        (x_ref, s0_ref, s1_ref, cnt_ref, W2_ref, b2_ref, M1a_ref, M1b_ref,
         bm1_ref, g_ref, bln_ref, M2_ref, bm2_ref) = refs[:13]
        prefs = refs[13:13 + 2 * ne]
        outs = refs[13 + 2 * ne:]
        cntv = cnt_ref[...]
        seg_ = jnp.concatenate([s0_ref[...], s1_ref[...]], axis=1)
        msg = (jnp.dot(seg_ / jnp.maximum(cntv, 1.0), W2_ref[...],
                       preferred_element_type=jnp.float32)
               + b2_ref[...] * (cntv > 0))
        t = (jnp.dot(x_ref[...], M1a_ref[...],
                     preferred_element_type=jnp.float32)
             + jnp.dot(msg, M1b_ref[...], preferred_element_type=jnp.float32)
             + bm1_ref[...])
        mu = jnp.mean(t, axis=-1, keepdims=True)
        var = jnp.mean((t - mu) ** 2, axis=-1, keepdims=True)
        h = jnp.maximum(
            g_ref[...] * (t - mu) / jnp.sqrt(var + 1e-5) + bln_ref[...], 0.0)
        y = jnp.dot(h, M2_ref[...],
                    preferred_element_type=jnp.float32) + bm2_ref[...]
        outs[0][...] = y
        for k in range(ne):
            outs[1 + k][...] = jnp.dot(
                y, prefs[2 * k][...],
                preferred_element_type=jnp.float32) + prefs[2 * k + 1][...]

    in_specs = [_rows(H), _rows(HH), _rows(HH), pl.BlockSpec((_RB, 1), lambda i: (i, 0)),
                _full((H, H)), _full((1, H)), _full((H, H)), _full((H, H)),
                _full((1, H)), _full((1, H)), _full((1, H)), _full((H, H)),
                _full((1, H))]
    args = [x, seg[0], seg[1], cnt, W2, b2.reshape(1, H), M1[:H], M1[H:],
            bm1.reshape(1, H), g.reshape(1, H), bln.reshape(1, H), M2,
            bm2.reshape(1, H)]
    for (P, p) in extras:
        in_specs += [_full((H, H)), _full((1, H))]
        args += [P, p.reshape(1, H)]
    out_shapes = tuple(jax.ShapeDtypeStruct((NN, H), jnp.float32)
                       for _ in range(1 + ne))
    out_specs = tuple(_rows(H) for _ in range(1 + ne))
    return pl.pallas_call(
        body, grid=(_NG,), in_specs=in_specs, out_specs=out_specs,
        out_shape=out_shapes)(*args)


def _split_cols(t):
    return jnp.concatenate([t[:, :HH], t[:, HH:]], axis=0)


def _edge_phase(tableD, tableS, wa, dstg, srcg, agg3, ea3):
    seg2 = _make_edge_seg_kernel()(_split_cols(tableD), _split_cols(tableS),
                                   dstg, srcg, agg3, ea3, wa)
    return seg2


def kernel(constraint_features, variable_features, edge_index, edge_attr,
           params):
    src = edge_index[0].astype(jnp.int32)
    dst = edge_index[1].astype(jnp.int32)
    ea = edge_attr.astype(jnp.float32)

    dstg = jnp.concatenate([dst, dst + NN])
    srcg = jnp.concatenate([src, src + NN])
    dst3 = dst.reshape(NROW // CPS, CPS, K)
    src3 = src.reshape(NROW // CPS, CPS, K)
    ea3 = ea.reshape(NROW // CPS, CPS, K)

    cnts = _make_count_kernel()(
        jnp.concatenate([dst3, src3]))
    cnt_c = cnts[0, :, 0:1]
    cnt_v = cnts[1, :, 0:1]

    lp0, lp1 = params['layers']
    W1_0 = lp0['cmsg1']['W']
    V1_0 = lp0['vmsg1']['W']
    W1_1 = lp1['cmsg1']['W']
    V1_1 = lp1['vmsg1']['W']

    cf = jnp.pad(constraint_features, ((0, 0), (0, 3)))
    vf = jnp.pad(variable_features, ((0, 0), (0, 5)))
    Wc = jnp.pad(params['cin']['W'], ((0, 3), (0, 0)))
    Wv = jnp.pad(params['vin']['W'], ((0, 5), (0, 0)))
    ch, A1 = _project(cf, Wc, params['cin']['b'],
                      [(W1_0[:H], jnp.zeros((H,), jnp.float32))])
    vh, B1, B2 = _project(
        vf, Wv, params['vin']['b'],
        [(W1_0[H:2 * H], lp0['cmsg1']['b']),
         (V1_0[:H], lp0['vmsg1']['b'])])

    seg = _edge_phase(A1, B1, W1_0[2 * H], dstg, srcg, dst3, ea3)
    ch, A2, A1n = _node_update(
        ch, seg, cnt_c, lp0['cmsg2']['W'], lp0['cmsg2']['b'],
        lp0['cmlp1']['W'], lp0['cmlp1']['b'], lp0['cln_g'], lp0['cln_b'],
        lp0['cmlp2']['W'], lp0['cmlp2']['b'],
        [(V1_0[H:2 * H], jnp.zeros((H,), jnp.float32)),
         (W1_1[:H], jnp.zeros((H,), jnp.float32))])

    seg = _edge_phase(A2, B2, V1_0[2 * H], dstg, srcg, src3, ea3)
    vh, B1n, B2n = _node_update(
        vh, seg, cnt_v, lp0['vmsg2']['W'], lp0['vmsg2']['b'],
        lp0['vmlp1']['W'], lp0['vmlp1']['b'], lp0['vln_g'], lp0['vln_b'],
        lp0['vmlp2']['W'], lp0['vmlp2']['b'],
        [(W1_1[H:2 * H], lp1['cmsg1']['b']),
         (V1_1[:H], lp1['vmsg1']['b'])])

    seg = _edge_phase(A1n, B1n, W1_1[2 * H], dstg, srcg, dst3, ea3)
    ch, A2n, out_c = _node_update(
        ch, seg, cnt_c, lp1['cmsg2']['W'], lp1['cmsg2']['b'],
        lp1['cmlp1']['W'], lp1['cmlp1']['b'], lp1['cln_g'], lp1['cln_b'],
        lp1['cmlp2']['W'], lp1['cmlp2']['b'],
        [(V1_1[H:2 * H], jnp.zeros((H,), jnp.float32)),
         (params['cout']['W'], params['cout']['b'])])

    seg = _edge_phase(A2n, B2n, V1_1[2 * H], dstg, srcg, src3, ea3)
    _, out_v = _node_update(
        vh, seg, cnt_v, lp1['vmsg2']['W'], lp1['vmsg2']['b'],
        lp1['vmlp1']['W'], lp1['vmlp1']['b'], lp1['vln_g'], lp1['vln_b'],
        lp1['vmlp2']['W'], lp1['vmlp2']['b'],
        [(params['vout']['W'], params['vout']['b'])])

    return out_c, out_v
